# Initial kernel scaffold; baseline (speedup 1.0000x reference)
import jax
import jax.numpy as jnp
from jax import lax
from jax.experimental import pallas as pl
from jax.experimental.pallas import tpu as pltpu
from jax.experimental.pallas import tpu_sc as plsc

N_NODES = 10000
D = 128
W_AUG = 144

NC = 2
NS = 16
NW = NC * NS

CHUNK = 128
CHUNKS_PER_W = 79
E_PAD = NW * CHUNKS_PER_W * CHUNK
N_ACC = 10016
ROWS_PER_TILE = N_ACC // NS


def _sc_body(feat_hbm, src_hbm, dst_hbm, out_hbm, zbuf, rows, sidx, didx,
             acc, sem):
    c = lax.axis_index("c")
    s = lax.axis_index("s")
    wid = s * NC + c

    zero16 = jnp.zeros((16,), jnp.float32)

    def zrow(i, _):
        def zcol(j, _):
            zbuf[i, pl.ds(j * 16, 16)] = zero16
            return ()
        return lax.fori_loop(0, W_AUG // 16, zcol, ())

    lax.fori_loop(0, CHUNK, zrow, ())

    row0 = s * ROWS_PER_TILE
    for k in range(ROWS_PER_TILE // CHUNK):
        pltpu.sync_copy(zbuf, acc.at[pl.ds(row0 + k * CHUNK, CHUNK)])
    rem = ROWS_PER_TILE % CHUNK
    if rem:
        base = row0 + (ROWS_PER_TILE // CHUNK) * CHUNK
        pltpu.sync_copy(zbuf.at[pl.ds(0, rem)], acc.at[pl.ds(base, rem)])

    plsc.subcore_barrier()

    ebase = wid * (CHUNKS_PER_W * CHUNK)

    def step(g, _):
        off = ebase + g * CHUNK
        pltpu.sync_copy(src_hbm.at[pl.ds(off, CHUNK)], sidx)
        pltpu.sync_copy(dst_hbm.at[pl.ds(off, CHUNK)], didx)
        pltpu.async_copy(feat_hbm.at[sidx], rows, sem).wait()
        pltpu.sync_copy(rows, acc.at[didx], add=True)
        return ()

    lax.fori_loop(0, CHUNKS_PER_W, step, ())

    plsc.subcore_barrier()

    pltpu.sync_copy(acc.at[pl.ds(row0, ROWS_PER_TILE)],
                    out_hbm.at[c, pl.ds(row0, ROWS_PER_TILE)])


@jax.jit
def _sc_partials(feat_aug, src_p, dst_p):
    mesh = plsc.VectorSubcoreMesh(core_axis_name="c", subcore_axis_name="s")
    return pl.kernel(
        _sc_body,
        out_type=jax.ShapeDtypeStruct((NC, N_ACC, W_AUG), jnp.float32),
        mesh=mesh,
        scratch_types=[
            pltpu.VMEM((CHUNK, W_AUG), jnp.float32),
            pltpu.VMEM((CHUNK, W_AUG), jnp.float32),
            pltpu.VMEM((CHUNK,), jnp.int32),
            pltpu.VMEM((CHUNK,), jnp.int32),
            pltpu.VMEM_SHARED((N_ACC, W_AUG), jnp.float32),
            pltpu.SemaphoreType.DMA,
        ],
    )(feat_aug, src_p, dst_p)


RB = 400


def _tc_body(p_ref, wgc_ref, bgc_ref, wlt_ref, bl_ref, o_ref):
    x = p_ref[...]
    st = x[0] + x[1]
    deg = jnp.clip(st[:, D], 1.0, None)
    h = st[:, :D] / deg[:, None]
    ge = jnp.dot(h, wgc_ref[...], preferred_element_type=jnp.float32)
    ge = jnp.maximum(ge + bgc_ref[...], 0.0)
    top = jnp.max(ge, axis=1, keepdims=True)
    cnt = jnp.sum((ge == top).astype(jnp.float32), axis=1, keepdims=True)
    h2 = h * cnt
    o_ref[...] = (jnp.dot(h2, wlt_ref[...], preferred_element_type=jnp.float32)
                  + bl_ref[...])


@jax.jit
def _tc_finish(parts, W_gc, b_gc, W_lin_t, b_lin2d):
    grid = N_NODES // RB
    return pl.pallas_call(
        _tc_body,
        grid=(grid,),
        in_specs=[
            pl.BlockSpec((NC, RB, W_AUG), lambda i: (0, i, 0)),
            pl.BlockSpec((D, 3), lambda i: (0, 0)),
            pl.BlockSpec((1, 3), lambda i: (0, 0)),
            pl.BlockSpec((D, D), lambda i: (0, 0)),
            pl.BlockSpec((1, D), lambda i: (0, 0)),
        ],
        out_specs=pl.BlockSpec((RB, D), lambda i: (i, 0)),
        out_shape=jax.ShapeDtypeStruct((N_NODES, D), jnp.float32),
    )(parts, W_gc, b_gc, W_lin_t, b_lin2d)


def kernel(feature, edge_index, W_gc, b_gc, W_lin, b_lin):
    src = edge_index[0].astype(jnp.int32)
    dst = edge_index[1].astype(jnp.int32)
    e = src.shape[0]
    pad = E_PAD - e
    src_p = jnp.concatenate([src, jnp.zeros((pad,), jnp.int32)])
    dst_p = jnp.concatenate([dst, jnp.full((pad,), N_NODES, jnp.int32)])
    feat_aug = jnp.concatenate(
        [feature,
         jnp.ones((N_NODES, 1), jnp.float32),
         jnp.zeros((N_NODES, W_AUG - D - 1), jnp.float32)], axis=1)
    parts = _sc_partials(feat_aug, src_p, dst_p)
    return _tc_finish(parts, W_gc, b_gc, W_lin.T, b_lin.reshape(1, D))


# speedup vs baseline: 4.6676x; 4.6676x over previous
import jax
import jax.numpy as jnp
from jax import lax
from jax.experimental import pallas as pl
from jax.experimental.pallas import tpu as pltpu
from jax.experimental.pallas import tpu_sc as plsc

N_NODES = 10000
D = 128
W_AUG = 144

NC = 2
NS = 16
NW = NC * NS

CHUNK = 128
CHUNKS_PER_W = 79
E_PAD = NW * CHUNKS_PER_W * CHUNK
N_ACC = 10016
ROWS_PER_TILE = N_ACC // NS


def _sc_body(feat_hbm, src_hbm, dst_hbm, out_hbm, zbuf, rows, sidx, didx,
             acc, sem):
    c = lax.axis_index("c")
    s = lax.axis_index("s")
    wid = s * NC + c

    zero16 = jnp.zeros((16,), jnp.float32)

    def zrow(i, _):
        def zcol(j, _):
            zbuf[i, pl.ds(j * 16, 16)] = zero16
            return ()
        return lax.fori_loop(0, W_AUG // 16, zcol, ())

    lax.fori_loop(0, CHUNK, zrow, ())

    row0 = s * ROWS_PER_TILE
    for k in range(ROWS_PER_TILE // CHUNK):
        pltpu.sync_copy(zbuf, acc.at[pl.ds(row0 + k * CHUNK, CHUNK)])
    rem = ROWS_PER_TILE % CHUNK
    if rem:
        base = row0 + (ROWS_PER_TILE // CHUNK) * CHUNK
        pltpu.sync_copy(zbuf.at[pl.ds(0, rem)], acc.at[pl.ds(base, rem)])

    plsc.subcore_barrier()

    ebase = wid * (CHUNKS_PER_W * CHUNK)

    def step(g, _):
        off = ebase + g * CHUNK
        pltpu.sync_copy(src_hbm.at[pl.ds(off, CHUNK)], sidx)
        pltpu.sync_copy(dst_hbm.at[pl.ds(off, CHUNK)], didx)
        pltpu.async_copy(feat_hbm.at[sidx], rows, sem).wait()
        pltpu.sync_copy(rows, acc.at[didx], add=True)
        return ()

    lax.fori_loop(0, CHUNKS_PER_W, step, ())

    plsc.subcore_barrier()

    pltpu.sync_copy(acc.at[pl.ds(row0, ROWS_PER_TILE)],
                    out_hbm.at[c, pl.ds(row0, ROWS_PER_TILE)])


@jax.jit
def _sc_partials(feat_aug, src_p, dst_p):
    mesh = plsc.VectorSubcoreMesh(core_axis_name="c", subcore_axis_name="s")
    return pl.kernel(
        _sc_body,
        out_type=jax.ShapeDtypeStruct((NC, N_ACC, W_AUG), jnp.float32),
        mesh=mesh,
        scratch_types=[
            pltpu.VMEM((CHUNK, W_AUG), jnp.float32),
            pltpu.VMEM((CHUNK, W_AUG), jnp.float32),
            pltpu.VMEM((CHUNK,), jnp.int32),
            pltpu.VMEM((CHUNK,), jnp.int32),
            pltpu.VMEM_SHARED((N_ACC, W_AUG), jnp.float32),
            pltpu.SemaphoreType.DMA,
        ],
        compiler_params=pltpu.CompilerParams(use_tc_tiling_on_sc=False),
    )(feat_aug, src_p, dst_p)


RB = 400


def _tc_body(p_ref, wgc_ref, bgc_ref, wlt_ref, bl_ref, o_ref):
    x = p_ref[...]
    st = x[0] + x[1]
    deg = jnp.clip(st[:, D], 1.0, None)
    h = st[:, :D] / deg[:, None]
    ge = jnp.dot(h, wgc_ref[...], preferred_element_type=jnp.float32)
    ge = jnp.maximum(ge + bgc_ref[...], 0.0)
    top = jnp.max(ge, axis=1, keepdims=True)
    cnt = jnp.sum((ge == top).astype(jnp.float32), axis=1, keepdims=True)
    h2 = h * cnt
    o_ref[...] = (jnp.dot(h2, wlt_ref[...], preferred_element_type=jnp.float32)
                  + bl_ref[...])


@jax.jit
def _tc_finish(parts, W_gc, b_gc, W_lin_t, b_lin2d):
    grid = N_NODES // RB
    return pl.pallas_call(
        _tc_body,
        grid=(grid,),
        in_specs=[
            pl.BlockSpec((NC, RB, W_AUG), lambda i: (0, i, 0)),
            pl.BlockSpec((D, 3), lambda i: (0, 0)),
            pl.BlockSpec((1, 3), lambda i: (0, 0)),
            pl.BlockSpec((D, D), lambda i: (0, 0)),
            pl.BlockSpec((1, D), lambda i: (0, 0)),
        ],
        out_specs=pl.BlockSpec((RB, D), lambda i: (i, 0)),
        out_shape=jax.ShapeDtypeStruct((N_NODES, D), jnp.float32),
    )(parts, W_gc, b_gc, W_lin_t, b_lin2d)


def kernel(feature, edge_index, W_gc, b_gc, W_lin, b_lin):
    src = edge_index[0].astype(jnp.int32)
    dst = edge_index[1].astype(jnp.int32)
    e = src.shape[0]
    pad = E_PAD - e
    src_p = jnp.concatenate([src, jnp.zeros((pad,), jnp.int32)])
    dst_p = jnp.concatenate([dst, jnp.full((pad,), N_NODES, jnp.int32)])
    feat_aug = jnp.concatenate(
        [feature,
         jnp.ones((N_NODES, 1), jnp.float32),
         jnp.zeros((N_NODES, W_AUG - D - 1), jnp.float32)], axis=1)
    parts = _sc_partials(feat_aug, src_p, dst_p)
    return _tc_finish(parts, W_gc, b_gc, W_lin.T, b_lin.reshape(1, D))
